# baseline (speedup 1.0000x reference)
import functools

import jax
import jax.numpy as jnp
from jax import lax
from jax.experimental import pallas as pl
from jax.experimental.pallas import tpu as pltpu
from jax.experimental.pallas import tpu_sc as plsc

NC, NS, L = 2, 16, 16
NW = NC * NS
NBUF = 4
ROWS_PER_CHUNK = 2


def _sc_kernel(B, N, K, D):
    rows_per_w = N // NW
    chunks_per_b = rows_per_w // ROWS_PER_CHUNK
    total = B * chunks_per_b
    mesh = plsc.VectorSubcoreMesh(core_axis_name="c", subcore_axis_name="s")

    @functools.partial(
        pl.kernel,
        mesh=mesh,
        out_type=jax.ShapeDtypeStruct((B, N, K, D), jnp.float32),
        scratch_types=[
            pltpu.VMEM((rows_per_w, D), jnp.float32),
            pltpu.VMEM((NBUF, ROWS_PER_CHUNK, K, D), jnp.float32),
            pltpu.VMEM((NBUF, ROWS_PER_CHUNK, K, D), jnp.float32),
            pltpu.SemaphoreType.DMA((NBUF,)),
            pltpu.SemaphoreType.DMA((NBUF,)),
        ],
    )
    def body(x_hbm, e_hbm, o_hbm, ebuf, ibufs, obufs, in_sems, out_sems):
        c = lax.axis_index("c")
        s = lax.axis_index("s")
        wid = s * NC + c
        n0 = wid * rows_per_w

        pltpu.sync_copy(e_hbm.at[pl.ds(n0, rows_per_w)], ebuf)

        def in_copy(t, slot):
            bt = lax.div(t, chunks_per_b)
            j = lax.rem(t, chunks_per_b)
            return pltpu.make_async_copy(
                x_hbm.at[bt, pl.ds(n0 + j * ROWS_PER_CHUNK, ROWS_PER_CHUNK)],
                ibufs.at[slot],
                in_sems.at[slot],
            )

        def out_copy(t, slot):
            bt = lax.div(t, chunks_per_b)
            j = lax.rem(t, chunks_per_b)
            return pltpu.make_async_copy(
                obufs.at[slot],
                o_hbm.at[bt, pl.ds(n0 + j * ROWS_PER_CHUNK, ROWS_PER_CHUNK)],
                out_sems.at[slot],
            )

        for t in range(NBUF):
            in_copy(jnp.int32(t), t).start()

        def group(g, _):
            for b in range(NBUF):
                t = g * NBUF + b

                in_copy(t, b).wait()

                @pl.when(g >= 1)
                def _():
                    out_copy(t - NBUF, b).wait()

                j = lax.rem(t, chunks_per_b)

                def dbody(di, _):
                    off = di * L
                    for nl in range(ROWS_PER_CHUNK):
                        ev = ebuf[j * ROWS_PER_CHUNK + nl, pl.ds(off, L)]
                        for k in range(K):
                            obufs[b, nl, k, pl.ds(off, L)] = (
                                ibufs[b, nl, k, pl.ds(off, L)] + ev
                            )
                    return 0

                lax.fori_loop(0, D // L, dbody, 0)

                out_copy(t, b).start()

                @pl.when(t + NBUF < total)
                def _():
                    in_copy(t + NBUF, b).start()
            return 0

        lax.fori_loop(0, total // NBUF, group, 0)

        for b in range(NBUF):
            out_copy(jnp.int32(total - NBUF + b), b).wait()

    return body


def kernel(x, embedding):
    B, N, K, D = x.shape
    return _sc_kernel(B, N, K, D)(x, embedding)


# speedup vs baseline: 2.3981x; 2.3981x over previous
import jax
import jax.numpy as jnp
from jax import lax
from jax.experimental import pallas as pl
from jax.experimental.pallas import tpu as pltpu


def _make_body(B, N, K, D, NBUF, n_c):
    n_per_b = N // n_c
    total = B * n_per_b

    def body(x_hbm, e_ref, o_hbm, ibufs, obufs, in_sems, out_sems):
        def in_copy(t, slot):
            bt = lax.div(t, n_per_b)
            j = lax.rem(t, n_per_b)
            return pltpu.make_async_copy(
                x_hbm.at[bt, pl.ds(j * n_c, n_c)],
                ibufs.at[slot],
                in_sems.at[slot],
            )

        def out_copy(t, slot):
            bt = lax.div(t, n_per_b)
            j = lax.rem(t, n_per_b)
            return pltpu.make_async_copy(
                obufs.at[slot],
                o_hbm.at[bt, pl.ds(j * n_c, n_c)],
                out_sems.at[slot],
            )

        for t in range(NBUF):
            in_copy(jnp.int32(t), t).start()

        def step(c, _):
            slot = lax.rem(c, NBUF)
            j = lax.rem(c, n_per_b)

            in_copy(c, slot).wait()

            @pl.when(c >= NBUF)
            def _():
                out_copy(c - NBUF, slot).wait()

            e_blk = e_ref[pl.ds(j * n_c, n_c), :]
            obufs[slot] = ibufs[slot] + e_blk[:, None, :]

            out_copy(c, slot).start()

            @pl.when(c + NBUF < total)
            def _():
                in_copy(c + NBUF, slot).start()

            return 0

        lax.fori_loop(0, total, step, 0)

        for t in range(max(0, total - NBUF), total):
            out_copy(jnp.int32(t), t % NBUF).wait()

    return body


def kernel(x, embedding):
    B, N, K, D = x.shape
    NBUF = 16
    n_c = 64
    body = _make_body(B, N, K, D, NBUF, n_c)
    return pl.pallas_call(
        body,
        grid=(),
        in_specs=[
            pl.BlockSpec(memory_space=pl.ANY),
            pl.BlockSpec(memory_space=pltpu.MemorySpace.VMEM),
        ],
        out_specs=pl.BlockSpec(memory_space=pl.ANY),
        out_shape=jax.ShapeDtypeStruct(x.shape, x.dtype),
        scratch_shapes=[
            pltpu.VMEM((NBUF, n_c, K, D), jnp.float32),
            pltpu.VMEM((NBUF, n_c, K, D), jnp.float32),
            pltpu.SemaphoreType.DMA((NBUF,)),
            pltpu.SemaphoreType.DMA((NBUF,)),
        ],
        compiler_params=pltpu.CompilerParams(
            vmem_limit_bytes=100 * 1024 * 1024,
        ),
    )(x, embedding)
